# baseline (speedup 1.0000x reference)
import functools

import jax
import jax.numpy as jnp
from jax import lax
from jax.experimental import pallas as pl
from jax.experimental.pallas import tpu as pltpu
from jax.experimental.pallas import tpu_sc as plsc

_N = 10000
_E = 4000
_EW = 128
_RB = 200
_NBLK = _N // _RB


def _bloch_mats(theta):
    th = theta.reshape(15, 3)
    t, p, l = th[:, 0], th[:, 1], th[:, 2]
    ct, st = jnp.cos(t), jnp.sin(t)
    cp, sp = jnp.cos(p), jnp.sin(p)
    cl, sl = jnp.cos(l), jnp.sin(l)
    rows = [
        cp * ct * cl - sp * sl, -cp * ct * sl - sp * cl, cp * st,
        sp * ct * cl + cp * sl, -sp * ct * sl + cp * cl, sp * st,
        -st * cl, st * sl, ct,
    ]
    return jnp.stack(rows, axis=1).reshape(-1)


def _sc_body(x_hbm, ri_hbm, ro_hbm, b8_hbm, ro_buf, ri_buf, x_buf0, x_buf1,
             spill, sem0, sem1):
    wid = lax.axis_index("c") * 16 + lax.axis_index("s")
    e0 = (wid * _EW).astype(jnp.int32)

    def copies(blk, slot):
        sem = (sem0, sem1)[slot]
        r0 = blk * _RB
        return (
            pltpu.make_async_copy(
                ro_hbm.at[pl.ds(r0, _RB), pl.ds(e0, _EW)], ro_buf.at[slot], sem),
            pltpu.make_async_copy(
                ri_hbm.at[pl.ds(r0, _RB), pl.ds(e0, _EW)], ri_buf.at[slot], sem),
            pltpu.make_async_copy(
                x_hbm.at[pl.ds(r0 * 4, _RB * 4)],
                (x_buf0, x_buf1)[slot].at[pl.ds(0, _RB * 4)], sem),
        )

    zeros = jnp.zeros((16,), jnp.float32)
    for r in range(8):
        for c in range(_EW // 16):
            spill[r, pl.ds(c * 16, 16)] = zeros

    for d in copies(0, 0):
        d.start()
    for d in copies(1, 1):
        d.start()

    def outer(i, carry):
        for slot in range(2):
            blk = 2 * i + slot
            for d in copies(blk, slot):
                d.wait()
            rob = ro_buf.at[slot]
            rib = ri_buf.at[slot]
            for sb in range(2):
                acc = []
                for op in range(2):
                    for c in range(4):
                        for f in range(4):
                            acc.append(spill[4 * op + f,
                                             pl.ds(sb * 64 + c * 16, 16)])

                xb = (x_buf0, x_buf1)[slot]

                def rnd(v):
                    t = v * 65537.0
                    return t - (t - v)

                def row4(q, a, _sb=sb, _rob=rob, _rib=rib, _xb=xb):
                    a = list(a)
                    n0 = q * 4
                    xv = _xb[pl.ds(n0 * 4, 16)]
                    xv = rnd(xv)
                    for j in range(4):
                        n = n0 + j
                        ro = [_rob[n, pl.ds(_sb * 64 + c * 16, 16)]
                              for c in range(4)]
                        ri = [_rib[n, pl.ds(_sb * 64 + c * 16, 16)]
                              for c in range(4)]
                        ro = [rnd(v) for v in ro]
                        ri = [rnd(v) for v in ri]
                        for c in range(4):
                            for f in range(4):
                                xf = xv[4 * j + f]
                                a[c * 4 + f] = a[c * 4 + f] + xf * ro[c]
                                a[16 + c * 4 + f] = (a[16 + c * 4 + f]
                                                    + xf * ri[c])
                    return tuple(a)

                acc = plsc.parallel_loop(
                    0, _RB // 4, unroll=2, carry=tuple(acc))(row4)
                k = 0
                for op in range(2):
                    for c in range(4):
                        for f in range(4):
                            spill[4 * op + f,
                                  pl.ds(sb * 64 + c * 16, 16)] = acc[k]
                            k += 1
            nxt = blk + 2

            @pl.when(nxt < _NBLK)
            def _():
                for d in copies(nxt, slot):
                    d.start()
        return carry

    lax.fori_loop(0, _NBLK // 2, outer, 0)
    pltpu.sync_copy(spill, b8_hbm.at[:, pl.ds(e0, _EW)])


def _sc_matmul(X, Ri, Ro):
    mesh = plsc.VectorSubcoreMesh(core_axis_name="c", subcore_axis_name="s")
    run = functools.partial(
        pl.kernel,
        out_type=jax.ShapeDtypeStruct((8, _E), jnp.float32),
        mesh=mesh,
        scratch_types=[
            pltpu.VMEM((2, _RB, _EW), jnp.float32),
            pltpu.VMEM((2, _RB, _EW), jnp.float32),
            pltpu.VMEM((_RB * 4 + 16,), jnp.float32),
            pltpu.VMEM((_RB * 4 + 16,), jnp.float32),
            pltpu.VMEM((8, _EW), jnp.float32),
            pltpu.SemaphoreType.DMA,
            pltpu.SemaphoreType.DMA,
        ],
    )(_sc_body)
    return run(X.reshape(-1), Ri, Ro)


def _epilogue_kernel(coef, b_ref, out_ref):
    B = b_ref[...]
    s = jnp.sin(B)
    c = jnp.cos(B)

    def R(k, a, b):
        return coef[k * 9 + a * 3 + b]

    def leaf(i, k):
        si, ci = s[i:i + 1, :], c[i:i + 1, :]
        return (si * R(k, 0, 0) + ci * R(k, 0, 2),
                si * R(k, 1, 0) + ci * R(k, 1, 2),
                si * R(k, 2, 0) + ci * R(k, 2, 2))

    def leafz(i, k):
        return s[i:i + 1, :] * R(k, 2, 0) + c[i:i + 1, :] * R(k, 2, 2)

    def rot(k, r):
        x, y, z = r
        return (R(k, 0, 0) * x + R(k, 0, 1) * y + R(k, 0, 2) * z,
                R(k, 1, 0) * x + R(k, 1, 1) * y + R(k, 1, 2) * z,
                R(k, 2, 0) * x + R(k, 2, 1) * y + R(k, 2, 2) * z)

    def rotz(k, r):
        x, y, z = r
        return R(k, 2, 0) * x + R(k, 2, 1) * y + R(k, 2, 2) * z

    def chan(r, zc):
        return (r[0], zc * r[1], zc * r[2])

    z0 = leafz(0, 0)
    z1 = rotz(8, chan(leaf(1, 1), z0))
    z3 = leafz(3, 3)
    r2 = rot(9, chan(leaf(2, 2), z3))
    z2 = rotz(12, chan(r2, z1))
    z4 = leafz(4, 4)
    r5 = rot(10, chan(leaf(5, 5), z4))
    z7 = leafz(7, 7)
    z6 = rotz(11, chan(leaf(6, 6), z7))
    r5 = rot(13, chan(r5, z6))
    z5 = rotz(14, chan(r5, z2))
    out_ref[...] = (1.0 - z5) * 0.5


def _sc_full_unused(X, Ri, Ro, theta_learn):
    coef = _bloch_mats(theta_learn)
    b8 = _sc_matmul(X, Ri, Ro)
    out = pl.pallas_call(
        _epilogue_kernel,
        grid_spec=pltpu.PrefetchScalarGridSpec(
            num_scalar_prefetch=1,
            grid=(1,),
            in_specs=[pl.BlockSpec((8, _E), lambda i, c: (0, 0))],
            out_specs=pl.BlockSpec((1, _E), lambda i, c: (0, 0)),
        ),
        out_shape=jax.ShapeDtypeStruct((1, _E), jnp.float32),
    )(coef, b8)
    return out.reshape(_E)

_NB = 200
_EB = 4000
_NSTEPS = _N // (2 * _NB)
def _edge_net_kernel(coef, xa_ref, xb_ref, ria_ref, rib_ref, roa_ref, rob_ref,
                     out_ref, acc_ref):
    n = pl.program_id(1)
    n_steps = pl.num_programs(1)

    @pl.when(n == 0)
    def _init():
        acc_ref[...] = jnp.zeros_like(acc_ref)

    dimn = (((0,), (0,)), ((), ()))
    xa = xa_ref[...]
    xb = xb_ref[...]
    acc_ref[0:4, :] += (
        jax.lax.dot_general(xa, roa_ref[...], dimn,
                            preferred_element_type=jnp.float32)
        + jax.lax.dot_general(xb, rob_ref[...], dimn,
                              preferred_element_type=jnp.float32))
    acc_ref[4:8, :] += (
        jax.lax.dot_general(xa, ria_ref[...], dimn,
                            preferred_element_type=jnp.float32)
        + jax.lax.dot_general(xb, rib_ref[...], dimn,
                              preferred_element_type=jnp.float32))

    @pl.when(n == n_steps - 1)
    def _epilogue():
        B = acc_ref[...]
        s = jnp.sin(B)
        c = jnp.cos(B)

        def R(k, a, b):
            return coef[k * 9 + a * 3 + b]

        def leaf(i, k):
            si, ci = s[i:i + 1, :], c[i:i + 1, :]
            return (si * R(k, 0, 0) + ci * R(k, 0, 2),
                    si * R(k, 1, 0) + ci * R(k, 1, 2),
                    si * R(k, 2, 0) + ci * R(k, 2, 2))

        def leafz(i, k):
            return s[i:i + 1, :] * R(k, 2, 0) + c[i:i + 1, :] * R(k, 2, 2)

        def rot(k, r):
            x, y, z = r
            return (R(k, 0, 0) * x + R(k, 0, 1) * y + R(k, 0, 2) * z,
                    R(k, 1, 0) * x + R(k, 1, 1) * y + R(k, 1, 2) * z,
                    R(k, 2, 0) * x + R(k, 2, 1) * y + R(k, 2, 2) * z)

        def rotz(k, r):
            x, y, z = r
            return R(k, 2, 0) * x + R(k, 2, 1) * y + R(k, 2, 2) * z

        def chan(r, zc):
            return (r[0], zc * r[1], zc * r[2])

        z0 = leafz(0, 0)
        z1 = rotz(8, chan(leaf(1, 1), z0))
        z3 = leafz(3, 3)
        r2 = rot(9, chan(leaf(2, 2), z3))
        z2 = rotz(12, chan(r2, z1))
        z4 = leafz(4, 4)
        r5 = rot(10, chan(leaf(5, 5), z4))
        z7 = leafz(7, 7)
        z6 = rotz(11, chan(leaf(6, 6), z7))
        r5 = rot(13, chan(r5, z6))
        z5 = rotz(14, chan(r5, z2))
        out_ref[...] = (1.0 - z5) * 0.5


def _tc_kernel(X, Ri, Ro, theta_learn):
    coef = _bloch_mats(theta_learn)

    grid = (pl.cdiv(_E, _EB), _NSTEPS)
    out = pl.pallas_call(
        _edge_net_kernel,
        grid_spec=pltpu.PrefetchScalarGridSpec(
            num_scalar_prefetch=1,
            grid=grid,
            in_specs=[
                pl.BlockSpec((_NB, 4), lambda e, n, c: (n, 0)),
                pl.BlockSpec((_NB, 4), lambda e, n, c: (n + _NSTEPS, 0)),
                pl.BlockSpec((_NB, _EB), lambda e, n, c: (n, e)),
                pl.BlockSpec((_NB, _EB), lambda e, n, c: (n + _NSTEPS, e)),
                pl.BlockSpec((_NB, _EB), lambda e, n, c: (n, e)),
                pl.BlockSpec((_NB, _EB), lambda e, n, c: (n + _NSTEPS, e)),
            ],
            out_specs=pl.BlockSpec((1, _EB), lambda e, n, c: (0, e)),
            scratch_shapes=[pltpu.VMEM((8, _EB), jnp.float32)],
        ),
        out_shape=jax.ShapeDtypeStruct((1, _E), jnp.float32),
        compiler_params=pltpu.CompilerParams(
            dimension_semantics=("arbitrary", "arbitrary"),
        ),
    )(coef, X, X, Ri, Ri, Ro, Ro)
    return out.reshape(_E)


@jax.jit
def kernel(X, Ri, Ro, theta_learn):
    b8 = _sc_matmul(X, Ri, Ro)
    out = _tc_kernel(X, Ri, Ro, theta_learn)
    return out + b8[0, :_E] * 1e-38


# speedup vs baseline: 2.5804x; 2.5804x over previous
import functools

import jax
import jax.numpy as jnp
from jax import lax
from jax.experimental import pallas as pl
from jax.experimental.pallas import tpu as pltpu
from jax.experimental.pallas import tpu_sc as plsc

_N = 10000
_E = 4000

_N_SC = 2000
_EW = 128
_RB = 200
_NBLK = _N_SC // _RB

_N_TC0 = _N_SC
_NB = 200
_NSTEPS = (_N - _N_TC0) // (2 * _NB)


def _bloch_mats(theta):
    th = theta.reshape(15, 3)
    t, p, l = th[:, 0], th[:, 1], th[:, 2]
    ct, st = jnp.cos(t), jnp.sin(t)
    cp, sp = jnp.cos(p), jnp.sin(p)
    cl, sl = jnp.cos(l), jnp.sin(l)
    rows = [
        cp * ct * cl - sp * sl, -cp * ct * sl - sp * cl, cp * st,
        sp * ct * cl + cp * sl, -sp * ct * sl + cp * cl, sp * st,
        -st * cl, st * sl, ct,
    ]
    return jnp.stack(rows, axis=1).reshape(-1)


def _sc_body(x_hbm, ri_hbm, ro_hbm, b8_hbm, ro_buf, ri_buf, x_buf0, x_buf1,
             spill, sem0, sem1):
    wid = lax.axis_index("c") * 16 + lax.axis_index("s")
    e0 = (wid * _EW).astype(jnp.int32)

    def copies(blk, slot):
        sem = (sem0, sem1)[slot]
        r0 = blk * _RB
        return (
            pltpu.make_async_copy(
                ro_hbm.at[pl.ds(r0, _RB), pl.ds(e0, _EW)], ro_buf.at[slot], sem),
            pltpu.make_async_copy(
                ri_hbm.at[pl.ds(r0, _RB), pl.ds(e0, _EW)], ri_buf.at[slot], sem),
            pltpu.make_async_copy(
                x_hbm.at[pl.ds(r0 * 4, _RB * 4)],
                (x_buf0, x_buf1)[slot].at[pl.ds(0, _RB * 4)], sem),
        )

    zeros = jnp.zeros((16,), jnp.float32)
    for r in range(8):
        for c in range(_EW // 16):
            spill[r, pl.ds(c * 16, 16)] = zeros

    for d in copies(0, 0):
        d.start()
    for d in copies(1, 1):
        d.start()

    def outer(i, carry):
        for slot in range(2):
            blk = 2 * i + slot
            for d in copies(blk, slot):
                d.wait()
            rob = ro_buf.at[slot]
            rib = ri_buf.at[slot]
            for sb in range(2):
                acc = []
                for op in range(2):
                    for c in range(4):
                        for f in range(4):
                            acc.append(spill[4 * op + f,
                                             pl.ds(sb * 64 + c * 16, 16)])

                xb = (x_buf0, x_buf1)[slot]

                def rnd(v):
                    t = v * 65537.0
                    return t - (t - v)

                def row4(q, a, _sb=sb, _rob=rob, _rib=rib, _xb=xb):
                    a = list(a)
                    n0 = q * 4
                    xv = _xb[pl.ds(n0 * 4, 16)]
                    xv = rnd(xv)
                    for j in range(4):
                        n = n0 + j
                        ro = [_rob[n, pl.ds(_sb * 64 + c * 16, 16)]
                              for c in range(4)]
                        ri = [_rib[n, pl.ds(_sb * 64 + c * 16, 16)]
                              for c in range(4)]
                        ro = [rnd(v) for v in ro]
                        ri = [rnd(v) for v in ri]
                        for c in range(4):
                            for f in range(4):
                                xf = xv[4 * j + f]
                                a[c * 4 + f] = a[c * 4 + f] + xf * ro[c]
                                a[16 + c * 4 + f] = (a[16 + c * 4 + f]
                                                    + xf * ri[c])
                    return tuple(a)

                acc = plsc.parallel_loop(
                    0, _RB // 4, unroll=2, carry=tuple(acc))(row4)
                k = 0
                for op in range(2):
                    for c in range(4):
                        for f in range(4):
                            spill[4 * op + f,
                                  pl.ds(sb * 64 + c * 16, 16)] = acc[k]
                            k += 1
            nxt = blk + 2

            @pl.when(nxt < _NBLK)
            def _():
                for d in copies(nxt, slot):
                    d.start()
        return carry

    lax.fori_loop(0, _NBLK // 2, outer, 0)
    pltpu.sync_copy(spill, b8_hbm.at[:, pl.ds(e0, _EW)])


def _sc_matmul(X, Ri, Ro):
    mesh = plsc.VectorSubcoreMesh(core_axis_name="c", subcore_axis_name="s")
    run = functools.partial(
        pl.kernel,
        out_type=jax.ShapeDtypeStruct((8, _E), jnp.float32),
        mesh=mesh,
        scratch_types=[
            pltpu.VMEM((2, _RB, _EW), jnp.float32),
            pltpu.VMEM((2, _RB, _EW), jnp.float32),
            pltpu.VMEM((_RB * 4 + 16,), jnp.float32),
            pltpu.VMEM((_RB * 4 + 16,), jnp.float32),
            pltpu.VMEM((8, _EW), jnp.float32),
            pltpu.SemaphoreType.DMA,
            pltpu.SemaphoreType.DMA,
        ],
    )(_sc_body)
    return run(X.reshape(-1), Ri, Ro)


def _tc_matmul_kernel(xa_ref, xb_ref, ria_ref, rib_ref, roa_ref, rob_ref,
                      out_ref, acc_ref):
    n = pl.program_id(0)
    n_steps = pl.num_programs(0)

    @pl.when(n == 0)
    def _init():
        acc_ref[...] = jnp.zeros_like(acc_ref)

    dimn = (((0,), (0,)), ((), ()))
    xa = xa_ref[...]
    xb = xb_ref[...]
    acc_ref[0:4, :] += (
        jax.lax.dot_general(xa, roa_ref[...], dimn,
                            preferred_element_type=jnp.float32)
        + jax.lax.dot_general(xb, rob_ref[...], dimn,
                              preferred_element_type=jnp.float32))
    acc_ref[4:8, :] += (
        jax.lax.dot_general(xa, ria_ref[...], dimn,
                            preferred_element_type=jnp.float32)
        + jax.lax.dot_general(xb, rib_ref[...], dimn,
                              preferred_element_type=jnp.float32))

    @pl.when(n == n_steps - 1)
    def _done():
        out_ref[...] = acc_ref[...]


def _tc_matmul(X, Ri, Ro):
    base = _N_TC0 // _NB
    return pl.pallas_call(
        _tc_matmul_kernel,
        grid=(_NSTEPS,),
        in_specs=[
            pl.BlockSpec((_NB, 4), lambda n: (n + base, 0)),
            pl.BlockSpec((_NB, 4), lambda n: (n + base + _NSTEPS, 0)),
            pl.BlockSpec((_NB, _E), lambda n: (n + base, 0)),
            pl.BlockSpec((_NB, _E), lambda n: (n + base + _NSTEPS, 0)),
            pl.BlockSpec((_NB, _E), lambda n: (n + base, 0)),
            pl.BlockSpec((_NB, _E), lambda n: (n + base + _NSTEPS, 0)),
        ],
        out_specs=pl.BlockSpec((8, _E), lambda n: (0, 0)),
        scratch_shapes=[pltpu.VMEM((8, _E), jnp.float32)],
        out_shape=jax.ShapeDtypeStruct((8, _E), jnp.float32),
    )(X, X, Ri, Ri, Ro, Ro)


def _epilogue_kernel(coef, bsc_ref, btc_ref, out_ref):
    B = bsc_ref[...] + btc_ref[...]
    s = jnp.sin(B)
    c = jnp.cos(B)

    def R(k, a, b):
        return coef[k * 9 + a * 3 + b]

    def leaf(i, k):
        si, ci = s[i:i + 1, :], c[i:i + 1, :]
        return (si * R(k, 0, 0) + ci * R(k, 0, 2),
                si * R(k, 1, 0) + ci * R(k, 1, 2),
                si * R(k, 2, 0) + ci * R(k, 2, 2))

    def leafz(i, k):
        return s[i:i + 1, :] * R(k, 2, 0) + c[i:i + 1, :] * R(k, 2, 2)

    def rot(k, r):
        x, y, z = r
        return (R(k, 0, 0) * x + R(k, 0, 1) * y + R(k, 0, 2) * z,
                R(k, 1, 0) * x + R(k, 1, 1) * y + R(k, 1, 2) * z,
                R(k, 2, 0) * x + R(k, 2, 1) * y + R(k, 2, 2) * z)

    def rotz(k, r):
        x, y, z = r
        return R(k, 2, 0) * x + R(k, 2, 1) * y + R(k, 2, 2) * z

    def chan(r, zc):
        return (r[0], zc * r[1], zc * r[2])

    z0 = leafz(0, 0)
    z1 = rotz(8, chan(leaf(1, 1), z0))
    z3 = leafz(3, 3)
    r2 = rot(9, chan(leaf(2, 2), z3))
    z2 = rotz(12, chan(r2, z1))
    z4 = leafz(4, 4)
    r5 = rot(10, chan(leaf(5, 5), z4))
    z7 = leafz(7, 7)
    z6 = rotz(11, chan(leaf(6, 6), z7))
    r5 = rot(13, chan(r5, z6))
    z5 = rotz(14, chan(r5, z2))
    out_ref[...] = (1.0 - z5) * 0.5


@jax.jit
def kernel(X, Ri, Ro, theta_learn):
    coef = _bloch_mats(theta_learn)
    b8_sc = _sc_matmul(X, Ri, Ro)
    b8_tc = _tc_matmul(X, Ri, Ro)
    out = pl.pallas_call(
        _epilogue_kernel,
        grid_spec=pltpu.PrefetchScalarGridSpec(
            num_scalar_prefetch=1,
            grid=(1,),
            in_specs=[
                pl.BlockSpec((8, _E), lambda i, c: (0, 0)),
                pl.BlockSpec((8, _E), lambda i, c: (0, 0)),
            ],
            out_specs=pl.BlockSpec((1, _E), lambda i, c: (0, 0)),
        ),
        out_shape=jax.ShapeDtypeStruct((1, _E), jnp.float32),
    )(coef, b8_sc, b8_tc)
    return out.reshape(_E)


# speedup vs baseline: 3.3932x; 1.3150x over previous
import functools

import jax
import jax.numpy as jnp
from jax import lax
from jax.experimental import pallas as pl
from jax.experimental.pallas import tpu as pltpu
from jax.experimental.pallas import tpu_sc as plsc

_N = 10000
_E = 4000

_N_SC = 800
_EW = 128
_RB = 200
_NBLK = _N_SC // _RB

_N_TC0 = _N_SC
_NB = 200
_NSTEPS = (_N - _N_TC0) // (2 * _NB)


def _bloch_mats(theta):
    th = theta.reshape(15, 3)
    t, p, l = th[:, 0], th[:, 1], th[:, 2]
    ct, st = jnp.cos(t), jnp.sin(t)
    cp, sp = jnp.cos(p), jnp.sin(p)
    cl, sl = jnp.cos(l), jnp.sin(l)
    rows = [
        cp * ct * cl - sp * sl, -cp * ct * sl - sp * cl, cp * st,
        sp * ct * cl + cp * sl, -sp * ct * sl + cp * cl, sp * st,
        -st * cl, st * sl, ct,
    ]
    return jnp.stack(rows, axis=1).reshape(-1)


def _sc_body(x_hbm, ri_hbm, ro_hbm, b8_hbm, ro_buf, ri_buf, x_buf0, x_buf1,
             spill, sem0, sem1):
    wid = lax.axis_index("c") * 16 + lax.axis_index("s")
    e0 = (wid * _EW).astype(jnp.int32)

    def copies(blk, slot):
        sem = (sem0, sem1)[slot]
        r0 = blk * _RB
        return (
            pltpu.make_async_copy(
                ro_hbm.at[pl.ds(r0, _RB), pl.ds(e0, _EW)], ro_buf.at[slot], sem),
            pltpu.make_async_copy(
                ri_hbm.at[pl.ds(r0, _RB), pl.ds(e0, _EW)], ri_buf.at[slot], sem),
            pltpu.make_async_copy(
                x_hbm.at[pl.ds(r0 * 4, _RB * 4)],
                (x_buf0, x_buf1)[slot].at[pl.ds(0, _RB * 4)], sem),
        )

    zeros = jnp.zeros((16,), jnp.float32)
    for r in range(8):
        for c in range(_EW // 16):
            spill[r, pl.ds(c * 16, 16)] = zeros

    for d in copies(0, 0):
        d.start()
    for d in copies(1, 1):
        d.start()

    def outer(i, carry):
        for slot in range(2):
            blk = 2 * i + slot
            for d in copies(blk, slot):
                d.wait()
            rob = ro_buf.at[slot]
            rib = ri_buf.at[slot]
            for sb in range(2):
                acc = []
                for op in range(2):
                    for c in range(4):
                        for f in range(4):
                            acc.append(spill[4 * op + f,
                                             pl.ds(sb * 64 + c * 16, 16)])

                xb = (x_buf0, x_buf1)[slot]

                def rnd(v):
                    t = v * 65537.0
                    return t - (t - v)

                def row4(q, a, _sb=sb, _rob=rob, _rib=rib, _xb=xb):
                    a = list(a)
                    n0 = q * 4
                    xv = _xb[pl.ds(n0 * 4, 16)]
                    xv = rnd(xv)
                    for j in range(4):
                        n = n0 + j
                        ro = [_rob[n, pl.ds(_sb * 64 + c * 16, 16)]
                              for c in range(4)]
                        ri = [_rib[n, pl.ds(_sb * 64 + c * 16, 16)]
                              for c in range(4)]
                        ro = [rnd(v) for v in ro]
                        ri = [rnd(v) for v in ri]
                        for c in range(4):
                            for f in range(4):
                                xf = xv[4 * j + f]
                                a[c * 4 + f] = a[c * 4 + f] + xf * ro[c]
                                a[16 + c * 4 + f] = (a[16 + c * 4 + f]
                                                    + xf * ri[c])
                    return tuple(a)

                acc = plsc.parallel_loop(
                    0, _RB // 4, unroll=2, carry=tuple(acc))(row4)
                k = 0
                for op in range(2):
                    for c in range(4):
                        for f in range(4):
                            spill[4 * op + f,
                                  pl.ds(sb * 64 + c * 16, 16)] = acc[k]
                            k += 1
            nxt = blk + 2

            @pl.when(nxt < _NBLK)
            def _():
                for d in copies(nxt, slot):
                    d.start()
        return carry

    lax.fori_loop(0, _NBLK // 2, outer, 0)
    pltpu.sync_copy(spill, b8_hbm.at[:, pl.ds(e0, _EW)])


def _sc_matmul(X, Ri, Ro):
    mesh = plsc.VectorSubcoreMesh(core_axis_name="c", subcore_axis_name="s")
    run = functools.partial(
        pl.kernel,
        out_type=jax.ShapeDtypeStruct((8, _E), jnp.float32),
        mesh=mesh,
        scratch_types=[
            pltpu.VMEM((2, _RB, _EW), jnp.float32),
            pltpu.VMEM((2, _RB, _EW), jnp.float32),
            pltpu.VMEM((_RB * 4 + 16,), jnp.float32),
            pltpu.VMEM((_RB * 4 + 16,), jnp.float32),
            pltpu.VMEM((8, _EW), jnp.float32),
            pltpu.SemaphoreType.DMA,
            pltpu.SemaphoreType.DMA,
        ],
    )(_sc_body)
    return run(X.reshape(-1), Ri, Ro)


def _tc_matmul_kernel(xa_ref, xb_ref, ria_ref, rib_ref, roa_ref, rob_ref,
                      out_ref, acc_ref):
    n = pl.program_id(0)
    n_steps = pl.num_programs(0)

    @pl.when(n == 0)
    def _init():
        acc_ref[...] = jnp.zeros_like(acc_ref)

    dimn = (((0,), (0,)), ((), ()))
    xa = xa_ref[...]
    xb = xb_ref[...]
    acc_ref[0:4, :] += (
        jax.lax.dot_general(xa, roa_ref[...], dimn,
                            preferred_element_type=jnp.float32)
        + jax.lax.dot_general(xb, rob_ref[...], dimn,
                              preferred_element_type=jnp.float32))
    acc_ref[4:8, :] += (
        jax.lax.dot_general(xa, ria_ref[...], dimn,
                            preferred_element_type=jnp.float32)
        + jax.lax.dot_general(xb, rib_ref[...], dimn,
                              preferred_element_type=jnp.float32))

    @pl.when(n == n_steps - 1)
    def _done():
        out_ref[...] = acc_ref[...]


def _tc_matmul(X, Ri, Ro):
    base = _N_TC0 // _NB
    return pl.pallas_call(
        _tc_matmul_kernel,
        grid=(_NSTEPS,),
        in_specs=[
            pl.BlockSpec((_NB, 4), lambda n: (n + base, 0)),
            pl.BlockSpec((_NB, 4), lambda n: (n + base + _NSTEPS, 0)),
            pl.BlockSpec((_NB, _E), lambda n: (n + base, 0)),
            pl.BlockSpec((_NB, _E), lambda n: (n + base + _NSTEPS, 0)),
            pl.BlockSpec((_NB, _E), lambda n: (n + base, 0)),
            pl.BlockSpec((_NB, _E), lambda n: (n + base + _NSTEPS, 0)),
        ],
        out_specs=pl.BlockSpec((8, _E), lambda n: (0, 0)),
        scratch_shapes=[pltpu.VMEM((8, _E), jnp.float32)],
        out_shape=jax.ShapeDtypeStruct((8, _E), jnp.float32),
    )(X, X, Ri, Ri, Ro, Ro)


def _epilogue_kernel(coef, bsc_ref, btc_ref, out_ref):
    B = bsc_ref[...] + btc_ref[...]
    s = jnp.sin(B)
    c = jnp.cos(B)

    def R(k, a, b):
        return coef[k * 9 + a * 3 + b]

    def leaf(i, k):
        si, ci = s[i:i + 1, :], c[i:i + 1, :]
        return (si * R(k, 0, 0) + ci * R(k, 0, 2),
                si * R(k, 1, 0) + ci * R(k, 1, 2),
                si * R(k, 2, 0) + ci * R(k, 2, 2))

    def leafz(i, k):
        return s[i:i + 1, :] * R(k, 2, 0) + c[i:i + 1, :] * R(k, 2, 2)

    def rot(k, r):
        x, y, z = r
        return (R(k, 0, 0) * x + R(k, 0, 1) * y + R(k, 0, 2) * z,
                R(k, 1, 0) * x + R(k, 1, 1) * y + R(k, 1, 2) * z,
                R(k, 2, 0) * x + R(k, 2, 1) * y + R(k, 2, 2) * z)

    def rotz(k, r):
        x, y, z = r
        return R(k, 2, 0) * x + R(k, 2, 1) * y + R(k, 2, 2) * z

    def chan(r, zc):
        return (r[0], zc * r[1], zc * r[2])

    z0 = leafz(0, 0)
    z1 = rotz(8, chan(leaf(1, 1), z0))
    z3 = leafz(3, 3)
    r2 = rot(9, chan(leaf(2, 2), z3))
    z2 = rotz(12, chan(r2, z1))
    z4 = leafz(4, 4)
    r5 = rot(10, chan(leaf(5, 5), z4))
    z7 = leafz(7, 7)
    z6 = rotz(11, chan(leaf(6, 6), z7))
    r5 = rot(13, chan(r5, z6))
    z5 = rotz(14, chan(r5, z2))
    out_ref[...] = (1.0 - z5) * 0.5


@jax.jit
def kernel(X, Ri, Ro, theta_learn):
    coef = _bloch_mats(theta_learn)
    b8_sc = _sc_matmul(X, Ri, Ro)
    b8_tc = _tc_matmul(X, Ri, Ro)
    out = pl.pallas_call(
        _epilogue_kernel,
        grid_spec=pltpu.PrefetchScalarGridSpec(
            num_scalar_prefetch=1,
            grid=(1,),
            in_specs=[
                pl.BlockSpec((8, _E), lambda i, c: (0, 0)),
                pl.BlockSpec((8, _E), lambda i, c: (0, 0)),
            ],
            out_specs=pl.BlockSpec((1, _E), lambda i, c: (0, 0)),
        ),
        out_shape=jax.ShapeDtypeStruct((1, _E), jnp.float32),
    )(coef, b8_sc, b8_tc)
    return out.reshape(_E)


# speedup vs baseline: 3.4144x; 1.0063x over previous
import functools

import jax
import jax.numpy as jnp
from jax import lax
from jax.experimental import pallas as pl
from jax.experimental.pallas import tpu as pltpu
from jax.experimental.pallas import tpu_sc as plsc

_N = 10000
_E = 4000

_N_SC = 400
_EW = 128
_RB = 200
_NBLK = _N_SC // _RB

_N_TC0 = _N_SC
_NB = 200
_NSTEPS = (_N - _N_TC0) // (2 * _NB)


def _bloch_mats(theta):
    th = theta.reshape(15, 3)
    t, p, l = th[:, 0], th[:, 1], th[:, 2]
    ct, st = jnp.cos(t), jnp.sin(t)
    cp, sp = jnp.cos(p), jnp.sin(p)
    cl, sl = jnp.cos(l), jnp.sin(l)
    rows = [
        cp * ct * cl - sp * sl, -cp * ct * sl - sp * cl, cp * st,
        sp * ct * cl + cp * sl, -sp * ct * sl + cp * cl, sp * st,
        -st * cl, st * sl, ct,
    ]
    return jnp.stack(rows, axis=1).reshape(-1)


def _sc_body(x_hbm, ri_hbm, ro_hbm, b8_hbm, ro_buf, ri_buf, x_buf0, x_buf1,
             spill, sem0, sem1):
    wid = lax.axis_index("c") * 16 + lax.axis_index("s")
    e0 = (wid * _EW).astype(jnp.int32)

    def copies(blk, slot):
        sem = (sem0, sem1)[slot]
        r0 = blk * _RB
        return (
            pltpu.make_async_copy(
                ro_hbm.at[pl.ds(r0, _RB), pl.ds(e0, _EW)], ro_buf.at[slot], sem),
            pltpu.make_async_copy(
                ri_hbm.at[pl.ds(r0, _RB), pl.ds(e0, _EW)], ri_buf.at[slot], sem),
            pltpu.make_async_copy(
                x_hbm.at[pl.ds(r0 * 4, _RB * 4)],
                (x_buf0, x_buf1)[slot].at[pl.ds(0, _RB * 4)], sem),
        )

    zeros = jnp.zeros((16,), jnp.float32)
    for r in range(8):
        for c in range(_EW // 16):
            spill[r, pl.ds(c * 16, 16)] = zeros

    for d in copies(0, 0):
        d.start()
    for d in copies(1, 1):
        d.start()

    def outer(i, carry):
        for slot in range(2):
            blk = 2 * i + slot
            for d in copies(blk, slot):
                d.wait()
            rob = ro_buf.at[slot]
            rib = ri_buf.at[slot]
            for sb in range(2):
                acc = []
                for op in range(2):
                    for c in range(4):
                        for f in range(4):
                            acc.append(spill[4 * op + f,
                                             pl.ds(sb * 64 + c * 16, 16)])

                xb = (x_buf0, x_buf1)[slot]

                def rnd(v):
                    t = v * 65537.0
                    return t - (t - v)

                def row4(q, a, _sb=sb, _rob=rob, _rib=rib, _xb=xb):
                    a = list(a)
                    n0 = q * 4
                    xv = _xb[pl.ds(n0 * 4, 16)]
                    xv = rnd(xv)
                    for j in range(4):
                        n = n0 + j
                        ro = [_rob[n, pl.ds(_sb * 64 + c * 16, 16)]
                              for c in range(4)]
                        ri = [_rib[n, pl.ds(_sb * 64 + c * 16, 16)]
                              for c in range(4)]
                        ro = [rnd(v) for v in ro]
                        ri = [rnd(v) for v in ri]
                        for c in range(4):
                            for f in range(4):
                                xf = xv[4 * j + f]
                                a[c * 4 + f] = a[c * 4 + f] + xf * ro[c]
                                a[16 + c * 4 + f] = (a[16 + c * 4 + f]
                                                    + xf * ri[c])
                    return tuple(a)

                acc = plsc.parallel_loop(
                    0, _RB // 4, unroll=2, carry=tuple(acc))(row4)
                k = 0
                for op in range(2):
                    for c in range(4):
                        for f in range(4):
                            spill[4 * op + f,
                                  pl.ds(sb * 64 + c * 16, 16)] = acc[k]
                            k += 1
            nxt = blk + 2

            @pl.when(nxt < _NBLK)
            def _():
                for d in copies(nxt, slot):
                    d.start()
        return carry

    lax.fori_loop(0, _NBLK // 2, outer, 0)
    pltpu.sync_copy(spill, b8_hbm.at[:, pl.ds(e0, _EW)])


def _sc_matmul(X, Ri, Ro):
    mesh = plsc.VectorSubcoreMesh(core_axis_name="c", subcore_axis_name="s")
    run = functools.partial(
        pl.kernel,
        out_type=jax.ShapeDtypeStruct((8, _E), jnp.float32),
        mesh=mesh,
        scratch_types=[
            pltpu.VMEM((2, _RB, _EW), jnp.float32),
            pltpu.VMEM((2, _RB, _EW), jnp.float32),
            pltpu.VMEM((_RB * 4 + 16,), jnp.float32),
            pltpu.VMEM((_RB * 4 + 16,), jnp.float32),
            pltpu.VMEM((8, _EW), jnp.float32),
            pltpu.SemaphoreType.DMA,
            pltpu.SemaphoreType.DMA,
        ],
    )(_sc_body)
    return run(X.reshape(-1), Ri, Ro)


def _tc_matmul_kernel(xa_ref, xb_ref, ria_ref, rib_ref, roa_ref, rob_ref,
                      out_ref, acc_ref):
    n = pl.program_id(0)
    n_steps = pl.num_programs(0)

    @pl.when(n == 0)
    def _init():
        acc_ref[...] = jnp.zeros_like(acc_ref)

    dimn = (((0,), (0,)), ((), ()))
    xa = xa_ref[...]
    xb = xb_ref[...]
    acc_ref[0:4, :] += (
        jax.lax.dot_general(xa, roa_ref[...], dimn,
                            preferred_element_type=jnp.float32)
        + jax.lax.dot_general(xb, rob_ref[...], dimn,
                              preferred_element_type=jnp.float32))
    acc_ref[4:8, :] += (
        jax.lax.dot_general(xa, ria_ref[...], dimn,
                            preferred_element_type=jnp.float32)
        + jax.lax.dot_general(xb, rib_ref[...], dimn,
                              preferred_element_type=jnp.float32))

    @pl.when(n == n_steps - 1)
    def _done():
        out_ref[...] = acc_ref[...]


def _tc_matmul(X, Ri, Ro):
    base = _N_TC0 // _NB
    return pl.pallas_call(
        _tc_matmul_kernel,
        grid=(_NSTEPS,),
        in_specs=[
            pl.BlockSpec((_NB, 4), lambda n: (n + base, 0)),
            pl.BlockSpec((_NB, 4), lambda n: (n + base + _NSTEPS, 0)),
            pl.BlockSpec((_NB, _E), lambda n: (n + base, 0)),
            pl.BlockSpec((_NB, _E), lambda n: (n + base + _NSTEPS, 0)),
            pl.BlockSpec((_NB, _E), lambda n: (n + base, 0)),
            pl.BlockSpec((_NB, _E), lambda n: (n + base + _NSTEPS, 0)),
        ],
        out_specs=pl.BlockSpec((8, _E), lambda n: (0, 0)),
        scratch_shapes=[pltpu.VMEM((8, _E), jnp.float32)],
        out_shape=jax.ShapeDtypeStruct((8, _E), jnp.float32),
    )(X, X, Ri, Ri, Ro, Ro)


def _epilogue_kernel(coef, bsc_ref, btc_ref, out_ref):
    B = bsc_ref[...] + btc_ref[...]
    s = jnp.sin(B)
    c = jnp.cos(B)

    def R(k, a, b):
        return coef[k * 9 + a * 3 + b]

    def leaf(i, k):
        si, ci = s[i:i + 1, :], c[i:i + 1, :]
        return (si * R(k, 0, 0) + ci * R(k, 0, 2),
                si * R(k, 1, 0) + ci * R(k, 1, 2),
                si * R(k, 2, 0) + ci * R(k, 2, 2))

    def leafz(i, k):
        return s[i:i + 1, :] * R(k, 2, 0) + c[i:i + 1, :] * R(k, 2, 2)

    def rot(k, r):
        x, y, z = r
        return (R(k, 0, 0) * x + R(k, 0, 1) * y + R(k, 0, 2) * z,
                R(k, 1, 0) * x + R(k, 1, 1) * y + R(k, 1, 2) * z,
                R(k, 2, 0) * x + R(k, 2, 1) * y + R(k, 2, 2) * z)

    def rotz(k, r):
        x, y, z = r
        return R(k, 2, 0) * x + R(k, 2, 1) * y + R(k, 2, 2) * z

    def chan(r, zc):
        return (r[0], zc * r[1], zc * r[2])

    z0 = leafz(0, 0)
    z1 = rotz(8, chan(leaf(1, 1), z0))
    z3 = leafz(3, 3)
    r2 = rot(9, chan(leaf(2, 2), z3))
    z2 = rotz(12, chan(r2, z1))
    z4 = leafz(4, 4)
    r5 = rot(10, chan(leaf(5, 5), z4))
    z7 = leafz(7, 7)
    z6 = rotz(11, chan(leaf(6, 6), z7))
    r5 = rot(13, chan(r5, z6))
    z5 = rotz(14, chan(r5, z2))
    out_ref[...] = (1.0 - z5) * 0.5


@jax.jit
def kernel(X, Ri, Ro, theta_learn):
    coef = _bloch_mats(theta_learn)
    b8_sc = _sc_matmul(X, Ri, Ro)
    b8_tc = _tc_matmul(X, Ri, Ro)
    out = pl.pallas_call(
        _epilogue_kernel,
        grid_spec=pltpu.PrefetchScalarGridSpec(
            num_scalar_prefetch=1,
            grid=(1,),
            in_specs=[
                pl.BlockSpec((8, _E), lambda i, c: (0, 0)),
                pl.BlockSpec((8, _E), lambda i, c: (0, 0)),
            ],
            out_specs=pl.BlockSpec((1, _E), lambda i, c: (0, 0)),
        ),
        out_shape=jax.ShapeDtypeStruct((1, _E), jnp.float32),
    )(coef, b8_sc, b8_tc)
    return out.reshape(_E)
